# baseline (speedup 1.0000x reference)
import jax
import jax.numpy as jnp
from jax import lax
from jax.experimental import pallas as pl
from jax.experimental.pallas import tpu as pltpu
from jax.experimental.pallas import tpu_sc as plsc

VOCAB = 30522
HIDDEN = 768
MAXPOS = 512
B = 128
L = 512
EPS = 1e-12

NC, NS = 2, 16
NW = NC * NS
HB = B // 2
TOKH = HB * L
TPW = TOKH // NW
CH = 64
NCHUNK = TPW // CH

BB = 8


def _gather_body(ids_hbm, word_hbm, y_hbm,
                 ids_v, buf_a, buf_b, sem_a, sem_b, sem_oa, sem_ob):
    wid = lax.axis_index("s") * NC + lax.axis_index("c")
    base0 = wid * TPW
    pltpu.sync_copy(ids_hbm.at[pl.ds(base0, TPW)], ids_v)

    slots = ((buf_a, sem_a, sem_oa), (buf_b, sem_b, sem_ob))

    def issue(c, buf, sem_g):
        pltpu.async_copy(word_hbm.at[ids_v.at[pl.ds(c * CH, CH)]], buf, sem_g)

    for s in range(2):
        issue(s, slots[s][0], slots[s][1])

    def half(h, carry):
        for s in range(2):
            buf, sem_g, sem_o = slots[s]
            c = 2 * h + s
            pltpu.make_async_copy(word_hbm.at[pl.ds(0, CH)], buf,
                                  sem_g).wait()
            pltpu.async_copy(buf, y_hbm.at[pl.ds(base0 + c * CH, CH)], sem_o)
            pltpu.make_async_copy(buf, y_hbm.at[pl.ds(0, CH)], sem_o).wait()
            cn = jnp.minimum(c + 2, NCHUNK - 1)
            issue(cn, buf, sem_g)
        return carry

    lax.fori_loop(0, NCHUNK // 2, half, 0)
    for s in range(2):
        buf, sem_g, sem_o = slots[s]
        pltpu.make_async_copy(word_hbm.at[pl.ds(0, CH)], buf, sem_g).wait()


_gather_call = pl.kernel(
    _gather_body,
    out_type=jax.ShapeDtypeStruct((TOKH, HIDDEN), jnp.float32),
    mesh=plsc.VectorSubcoreMesh(core_axis_name="c", subcore_axis_name="s",
                                num_cores=NC, num_subcores=NS),
    scratch_types=[
        pltpu.VMEM((TPW,), jnp.int32),
        pltpu.VMEM((CH, HIDDEN), jnp.float32),
        pltpu.VMEM((CH, HIDDEN), jnp.float32),
        pltpu.SemaphoreType.DMA,
        pltpu.SemaphoreType.DMA,
        pltpu.SemaphoreType.DMA,
        pltpu.SemaphoreType.DMA,
    ],
    compiler_params=pltpu.CompilerParams(needs_layout_passes=False),
)


def _ln_math(x, ttf, type_ref, gamma_ref, beta_ref):
    t0 = type_ref[0, :]
    dt = type_ref[1, :] - t0
    x = x + t0[None, :] + ttf[:, None] * dt[None, :]
    mean = jnp.mean(x, axis=-1, keepdims=True)
    var = jnp.mean(jnp.square(x - mean), axis=-1, keepdims=True)
    x = (x - mean) * lax.rsqrt(var + EPS)
    return x * gamma_ref[...] + beta_ref[...]


def _ln_body0(y_ref, tt_ref, pos_ref, type_ref, gamma_ref, beta_ref,
              out_ref):
    for bb in range(BB):
        x = y_ref[bb] + pos_ref[...]
        out_ref[bb] = _ln_math(x, tt_ref[bb, 0, :], type_ref,
                               gamma_ref, beta_ref)


def _ln_body1(prev_ref, y_ref, tt_ref, pos_ref, type_ref, gamma_ref,
              beta_ref, out_ref):
    del prev_ref
    for bb in range(BB):
        x = y_ref[bb] + pos_ref[...]
        out_ref[bb] = _ln_math(x, tt_ref[bb, 0, :], type_ref,
                               gamma_ref, beta_ref)


_HGRID = HB // BB

_ln_call0 = pl.pallas_call(
    _ln_body0,
    grid=(_HGRID,),
    in_specs=[
        pl.BlockSpec((BB, L, HIDDEN), lambda b: (b, 0, 0)),
        pl.BlockSpec((BB, 1, L), lambda b: (b, 0, 0)),
        pl.BlockSpec((L, HIDDEN), lambda b: (0, 0)),
        pl.BlockSpec((2, HIDDEN), lambda b: (0, 0)),
        pl.BlockSpec((HIDDEN,), lambda b: (0,)),
        pl.BlockSpec((HIDDEN,), lambda b: (0,)),
    ],
    out_specs=pl.BlockSpec((BB, L, HIDDEN), lambda b: (b, 0, 0)),
    out_shape=jax.ShapeDtypeStruct((B, L, HIDDEN), jnp.float32),
    compiler_params=pltpu.CompilerParams(
        dimension_semantics=("arbitrary",)),
)

_ln_call1 = pl.pallas_call(
    _ln_body1,
    grid=(_HGRID,),
    in_specs=[
        pl.BlockSpec(memory_space=pl.ANY),
        pl.BlockSpec((BB, L, HIDDEN), lambda b: (b, 0, 0)),
        pl.BlockSpec((BB, 1, L), lambda b: (b, 0, 0)),
        pl.BlockSpec((L, HIDDEN), lambda b: (0, 0)),
        pl.BlockSpec((2, HIDDEN), lambda b: (0, 0)),
        pl.BlockSpec((HIDDEN,), lambda b: (0,)),
        pl.BlockSpec((HIDDEN,), lambda b: (0,)),
    ],
    out_specs=pl.BlockSpec((BB, L, HIDDEN), lambda b: (b + _HGRID, 0, 0)),
    out_shape=jax.ShapeDtypeStruct((B, L, HIDDEN), jnp.float32),
    input_output_aliases={0: 0},
    compiler_params=pltpu.CompilerParams(
        dimension_semantics=("arbitrary",)),
)


def kernel(input_ids, token_type_ids, word_emb, pos_emb, type_emb,
           ln_gamma, ln_beta):
    ids = input_ids.reshape(-1).astype(jnp.int32)
    ttf = token_type_ids.astype(jnp.float32).reshape(B, 1, L)
    y0 = _gather_call(ids[:TOKH], word_emb).reshape(HB, L, HIDDEN)
    y1 = _gather_call(ids[TOKH:], word_emb).reshape(HB, L, HIDDEN)
    o0 = _ln_call0(y0, ttf[:HB], pos_emb, type_emb, ln_gamma, ln_beta)
    return _ln_call1(o0, y1, ttf[HB:], pos_emb, type_emb, ln_gamma, ln_beta)
